# Initial kernel scaffold; baseline (speedup 1.0000x reference)
import jax
import jax.numpy as jnp
from jax.experimental import pallas as pl


def kernel(in_feat, edge_weights, W1, b1, W2, b2, Wd, bd, Wc, bc, edge_index):
    raise NotImplementedError("write your pallas kernel here")


# speedup vs baseline: 3.9108x; 3.9108x over previous
import functools

import jax
import jax.numpy as jnp
from jax import lax
from jax.experimental import pallas as pl
from jax.experimental.pallas import tpu as pltpu
from jax.experimental.pallas import tpu_sc as plsc

N = 10000
E = 160000
D = 256
HALF = 128
AUGW = 144
NC = 2
NS = 16
LANES = 16
B = 128
ROWS = E // B
ROWS_PT = ROWS // NS
ROWS_REM = ROWS - ROWS_PT * NS
NPT = N // NS
BLK = 1000
NBLK = N // BLK


def _make_sc_aggregate(width):
  mesh = plsc.VectorSubcoreMesh(
      core_axis_name="c", subcore_axis_name="s", num_cores=NC, num_subcores=NS)

  @functools.partial(
      pl.kernel,
      out_type=jax.ShapeDtypeStruct((NC * N, width), jnp.float32),
      mesh=mesh,
      scratch_types=[
          pltpu.VMEM_SHARED((N, width), jnp.float32),
          pltpu.VMEM((B, width), jnp.float32),
          pltpu.VMEM((B,), jnp.int32),
          pltpu.VMEM((B,), jnp.int32),
          pltpu.VMEM((B,), jnp.float32),
          pltpu.SemaphoreType.DMA,
      ],
      compiler_params=pltpu.CompilerParams(use_tc_tiling_on_sc=False),
  )
  def agg(table_hbm, src_hbm, dst_hbm, w_hbm, zeros_hbm, out_hbm,
          acc, rows, srcv, dstv, wv, sem):
    c = lax.axis_index("c")
    s = lax.axis_index("s")
    pltpu.sync_copy(zeros_hbm, acc.at[pl.ds(s * NPT, NPT)])
    plsc.subcore_barrier()

    start = s * ROWS_PT + jnp.minimum(s, ROWS_REM)
    count = ROWS_PT + jnp.where(s < ROWS_REM, 1, 0)
    off = c * N

    def batch_body(b, carry):
      r = start + b
      pltpu.sync_copy(src_hbm.at[r], srcv)
      pltpu.sync_copy(dst_hbm.at[r], dstv)
      pltpu.sync_copy(w_hbm.at[r], wv)
      for j in range(B // LANES):
        sl = pl.ds(j * LANES, LANES)
        srcv[sl] = srcv[sl] + off
      pltpu.async_copy(table_hbm.at[srcv], rows, sem).wait()

      def mul_chunk(kb, carry2):
        base = kb * LANES
        wk_vec = wv[pl.ds(base, LANES)]
        for l in range(LANES):
          wk = wk_vec[l]
          for j in range(width // LANES):
            sl = pl.ds(j * LANES, LANES)
            rows[base + l, sl] = rows[base + l, sl] * wk
        return carry2

      lax.fori_loop(0, B // LANES, mul_chunk, 0)
      pltpu.sync_copy(rows, acc.at[dstv], add=True)
      return carry

    lax.fori_loop(0, count, batch_body, 0)
    plsc.subcore_barrier()
    pltpu.sync_copy(acc.at[pl.ds(s * NPT, NPT)],
                    out_hbm.at[pl.ds(c * N + s * NPT, NPT)])

  return agg


_sc_agg_aug = _make_sc_aggregate(AUGW)
_sc_agg_half = _make_sc_aggregate(HALF)


def _tc_layer1(a1, w1, b1):
  def body(aa_ref, ab_ref, w1a_ref, w1b_ref, b1_ref, out_ref):
    aa = aa_ref[...]
    ab = ab_ref[...]
    deg = aa[:, HALF:HALF + 1]
    scale = jnp.where(deg > 0.0, 1.0 / deg, 0.0)
    xa = aa[:, :HALF] * scale
    xb = ab[:, :HALF] * scale
    h = (jnp.dot(xa, w1a_ref[...], preferred_element_type=jnp.float32)
         + jnp.dot(xb, w1b_ref[...], preferred_element_type=jnp.float32)
         + b1_ref[...])
    out_ref[...] = jnp.maximum(h, 0.0)

  return pl.pallas_call(
      body,
      grid=(2, NBLK),
      in_specs=[
          pl.BlockSpec((BLK, AUGW), lambda j, i: (i, 0)),
          pl.BlockSpec((BLK, AUGW), lambda j, i: (i + NBLK, 0)),
          pl.BlockSpec((HALF, HALF), lambda j, i: (0, j)),
          pl.BlockSpec((HALF, HALF), lambda j, i: (1, j)),
          pl.BlockSpec((1, HALF), lambda j, i: (0, j)),
      ],
      out_specs=pl.BlockSpec((BLK, HALF), lambda j, i: (j * NBLK + i, 0)),
      out_shape=jax.ShapeDtypeStruct((2 * N, HALF), jnp.float32),
      compiler_params=pltpu.CompilerParams(
          dimension_semantics=("parallel", "parallel")),
  )(a1, a1, w1, w1, b1.reshape(1, D))


def _tc_layer2(a2, a1, w2, b2, wd, bd, wc, bc):
  def body(a2a_ref, a2b_ref, dega_ref, w2a_ref, w2b_ref, b2_ref,
           wd_ref, bd_ref, wc_ref, bc_ref, out_ref, acc_ref):
    i = pl.program_id(0)

    @pl.when(i == 0)
    def _():
      acc_ref[...] = jnp.zeros_like(acc_ref)

    deg = dega_ref[...][:, HALF:HALF + 1]
    scale = jnp.where(deg > 0.0, 1.0 / deg, 0.0)
    xa = a2a_ref[...] * scale
    xb = a2b_ref[...] * scale
    h2 = (jnp.dot(xa, w2a_ref[...], preferred_element_type=jnp.float32)
          + jnp.dot(xb, w2b_ref[...], preferred_element_type=jnp.float32)
          + b2_ref[...])
    h2 = jnp.maximum(h2, 0.0)
    acc_ref[...] += jnp.sum(h2, axis=0, keepdims=True)

    @pl.when(i == NBLK - 1)
    def _():
      hg = acc_ref[...] * (1.0 / N)
      o1 = jnp.maximum(
          jnp.dot(hg, wd_ref[...], preferred_element_type=jnp.float32)
          + bd_ref[...], 0.0)
      out_ref[...] = (
          jnp.dot(o1, wc_ref[...], preferred_element_type=jnp.float32)
          + bc_ref[...])

  return pl.pallas_call(
      body,
      grid=(NBLK,),
      in_specs=[
          pl.BlockSpec((BLK, HALF), lambda i: (i, 0)),
          pl.BlockSpec((BLK, HALF), lambda i: (i + NBLK, 0)),
          pl.BlockSpec((BLK, AUGW), lambda i: (i, 0)),
          pl.BlockSpec((HALF, D), lambda i: (0, 0)),
          pl.BlockSpec((HALF, D), lambda i: (1, 0)),
          pl.BlockSpec((1, D), lambda i: (0, 0)),
          pl.BlockSpec((D, HALF), lambda i: (0, 0)),
          pl.BlockSpec((1, HALF), lambda i: (0, 0)),
          pl.BlockSpec((HALF, 10), lambda i: (0, 0)),
          pl.BlockSpec((1, 10), lambda i: (0, 0)),
      ],
      out_specs=pl.BlockSpec((1, 10), lambda i: (0, 0)),
      out_shape=jax.ShapeDtypeStruct((1, 10), jnp.float32),
      scratch_shapes=[pltpu.VMEM((1, D), jnp.float32)],
      compiler_params=pltpu.CompilerParams(
          dimension_semantics=("arbitrary",)),
  )(a2, a2, a1, w2, w2, b2.reshape(1, D), wd, bd.reshape(1, HALF),
    wc, bc.reshape(1, 10))


def kernel(in_feat, edge_weights, W1, b1, W2, b2, Wd, bd, Wc, bc, edge_index):
  src = edge_index[0].reshape(ROWS, B)
  dst = edge_index[1].reshape(ROWS, B)
  w = edge_weights.reshape(ROWS, B)

  ones = jnp.ones((N, 1), jnp.float32)
  pad = jnp.zeros((N, AUGW - HALF - 1), jnp.float32)
  table1 = jnp.concatenate([
      jnp.concatenate([in_feat[:, :HALF], ones, pad], axis=1),
      jnp.concatenate([in_feat[:, HALF:], ones, pad], axis=1),
  ], axis=0)

  zeros_aug = jnp.zeros((NPT, AUGW), jnp.float32)
  zeros_half = jnp.zeros((NPT, HALF), jnp.float32)

  a1 = _sc_agg_aug(table1, src, dst, w, zeros_aug)
  h = _tc_layer1(a1, W1, b1)
  a2 = _sc_agg_half(h, src, dst, w, zeros_half)
  return _tc_layer2(a2, a1, W2, b2, Wd, bd, Wc, bc)
